# baseline (speedup 1.0000x reference)
import functools

import jax
import jax.numpy as jnp
from jax import lax
from jax.experimental import pallas as pl
from jax.experimental.pallas import tpu as pltpu
from jax.experimental.pallas import tpu_sc as plsc

B = 64
Q = 16
KV = 32768
ROWS = B * Q

NUM_WORKERS = 32
ROWS_PER_WORKER = ROWS // NUM_WORKERS
LANES = 16
VECS = KV // LANES
GROUPS = ROWS_PER_WORKER // LANES


def _argmax_body(x_hbm, out_hbm, row_v, res_v, vs_v, es_v, sem):
    c = lax.axis_index("c")
    s = lax.axis_index("s")
    wid = s * 2 + c
    base = wid * ROWS_PER_WORKER

    lane_iota = lax.iota(jnp.int32, LANES)
    neg_inf = jnp.full((LANES,), -jnp.inf, dtype=jnp.float32)

    for g in range(GROUPS):
        def row_step(i, res_vec):
            r = base + g * LANES + i
            pltpu.sync_copy(x_hbm.at[r], row_v)

            def step(j, carry):
                bv, bp = carry
                v = row_v[pl.ds(j * LANES, LANES)]
                pred = v > bv
                bv = jnp.where(pred, v, bv)
                bp = jnp.where(pred, jnp.full((LANES,), j, jnp.int32), bp)
                return bv, bp

            bv, bp = lax.fori_loop(
                0, VECS, step,
                (neg_inf, jnp.zeros((LANES,), jnp.int32)),
                unroll=8,
            )
            v = bv
            e = bp * LANES + lane_iota
            for off in (1, 2, 4, 8):
                vs_v[...] = v
                es_v[...] = e
                perm = lane_iota ^ off
                ov = plsc.load_gather(vs_v, [perm])
                oe = plsc.load_gather(es_v, [perm])
                pred = (ov > v) | ((ov == v) & (oe < e))
                v = jnp.where(pred, ov, v)
                e = jnp.where(pred, oe, e)
            return jnp.where(lane_iota == i, e, res_vec)

        res_vec = lax.fori_loop(
            0, LANES, row_step, jnp.zeros((LANES,), jnp.int32))
        res_v[...] = res_vec
        pltpu.sync_copy(res_v, out_hbm.at[pl.ds(base + g * LANES, LANES)])


@jax.jit
def _argmax_rows(x):
    mesh = plsc.VectorSubcoreMesh(core_axis_name="c", subcore_axis_name="s")
    kern = functools.partial(
        pl.kernel,
        mesh=mesh,
        compiler_params=pltpu.CompilerParams(needs_layout_passes=False),
        out_type=jax.ShapeDtypeStruct((ROWS,), jnp.int32),
        scratch_types=[
            pltpu.VMEM((KV,), jnp.float32),
            pltpu.VMEM((LANES,), jnp.int32),
            pltpu.VMEM((LANES,), jnp.float32),
            pltpu.VMEM((LANES,), jnp.int32),
            pltpu.SemaphoreType.DMA,
        ],
    )(_argmax_body)
    return kern(x)


def kernel(xyz):
    x = xyz.reshape(ROWS, KV)
    out = _argmax_rows(x)
    return out.reshape(B, Q)


# speedup vs baseline: 1.5456x; 1.5456x over previous
import functools

import jax
import jax.numpy as jnp
from jax import lax
from jax.experimental import pallas as pl
from jax.experimental.pallas import tpu as pltpu
from jax.experimental.pallas import tpu_sc as plsc

B = 64
Q = 16
KV = 32768
ROWS = B * Q

NUM_WORKERS = 32
ROWS_PER_WORKER = ROWS // NUM_WORKERS
LANES = 16
VECS = KV // LANES
STREAMS = 4
SVECS = VECS // STREAMS


def _row_argmax(buf, lane_iota):
    neg_inf = jnp.full((LANES,), -jnp.inf, dtype=jnp.float32)
    zeros = jnp.zeros((LANES,), jnp.int32)

    def step(p, carry):
        pvec = jnp.full((LANES,), p, jnp.int32)
        new = []
        for s in range(STREAMS):
            bv, bp = carry[2 * s], carry[2 * s + 1]
            v = buf[pl.ds((s * SVECS + p) * LANES, LANES)]
            pred = v > bv
            new.append(jnp.where(pred, v, bv))
            new.append(jnp.where(pred, pvec, bp))
        return tuple(new)

    init = (neg_inf, zeros) * STREAMS
    acc = lax.fori_loop(0, SVECS, step, init, unroll=8)

    v, g = acc[0], acc[1]
    for s in range(1, STREAMS):
        sv, sp = acc[2 * s], acc[2 * s + 1] + jnp.int32(s * SVECS)
        pred = sv > v
        v = jnp.where(pred, sv, v)
        g = jnp.where(pred, sp, g)
    return v, g * LANES + lane_iota


def _lane_merge(v, e, vs_v, es_v, lane_iota):
    for off in (1, 2, 4, 8):
        vs_v[...] = v
        es_v[...] = e
        perm = lane_iota ^ off
        ov = plsc.load_gather(vs_v, [perm])
        oe = plsc.load_gather(es_v, [perm])
        pred = (ov > v) | ((ov == v) & (oe < e))
        v = jnp.where(pred, ov, v)
        e = jnp.where(pred, oe, e)
    return e


def _argmax_body(x_hbm, out_hbm, buf0, buf1, res_v, vs_v, es_v, sem0, sem1):
    c = lax.axis_index("c")
    s = lax.axis_index("s")
    wid = s * 2 + c
    base = wid * ROWS_PER_WORKER

    lane_iota = lax.iota(jnp.int32, LANES)

    def compute(buf, r_local):
        v, e = _row_argmax(buf, lane_iota)
        e = _lane_merge(v, e, vs_v, es_v, lane_iota)
        goff = (r_local // LANES) * LANES
        cur = res_v[pl.ds(goff, LANES)]
        res_v[pl.ds(goff, LANES)] = jnp.where(
            lane_iota == (r_local - goff), e, cur)

    pltpu.make_async_copy(x_hbm.at[base], buf0, sem0).start()

    def pair(p, _):
        r0 = 2 * p
        pltpu.make_async_copy(x_hbm.at[base + r0 + 1], buf1, sem1).start()
        pltpu.make_async_copy(x_hbm.at[base + r0], buf0, sem0).wait()
        compute(buf0, r0)

        @pl.when(p < ROWS_PER_WORKER // 2 - 1)
        def _():
            pltpu.make_async_copy(x_hbm.at[base + r0 + 2], buf0, sem0).start()

        pltpu.make_async_copy(x_hbm.at[base + r0 + 1], buf1, sem1).wait()
        compute(buf1, r0 + 1)
        return 0

    lax.fori_loop(0, ROWS_PER_WORKER // 2, pair, 0)

    pltpu.sync_copy(res_v, out_hbm.at[pl.ds(base, ROWS_PER_WORKER)])


@jax.jit
def _argmax_rows(x):
    mesh = plsc.VectorSubcoreMesh(core_axis_name="c", subcore_axis_name="s")
    kern = functools.partial(
        pl.kernel,
        mesh=mesh,
        compiler_params=pltpu.CompilerParams(needs_layout_passes=False),
        out_type=jax.ShapeDtypeStruct((ROWS,), jnp.int32),
        scratch_types=[
            pltpu.VMEM((KV,), jnp.float32),
            pltpu.VMEM((KV,), jnp.float32),
            pltpu.VMEM((ROWS_PER_WORKER,), jnp.int32),
            pltpu.VMEM((LANES,), jnp.float32),
            pltpu.VMEM((LANES,), jnp.int32),
            pltpu.SemaphoreType.DMA,
            pltpu.SemaphoreType.DMA,
        ],
    )(_argmax_body)
    return kern(x)


def kernel(xyz):
    x = xyz.reshape(ROWS, KV)
    out = _argmax_rows(x)
    return out.reshape(B, Q)


# speedup vs baseline: 2.0377x; 1.3184x over previous
import functools

import jax
import jax.numpy as jnp
from jax import lax
from jax.experimental import pallas as pl
from jax.experimental.pallas import tpu as pltpu
from jax.experimental.pallas import tpu_sc as plsc

B = 64
Q = 16
KV = 32768
ROWS = B * Q

NUM_WORKERS = 32
ROWS_PER_WORKER = ROWS // NUM_WORKERS
LANES = 16
VECS = KV // LANES
STREAMS = 4
SVECS = VECS // STREAMS


BLK = 16
NBLK = VECS // BLK
BIG = 2 ** 20
INT_MAX = 2 ** 31 - 1


def _bfly_max(v, vs_v, lane_iota):
    for off in (1, 2, 4, 8):
        vs_v[...] = v
        ov = plsc.load_gather(vs_v, [lane_iota ^ off])
        v = jnp.maximum(v, ov)
    return v


def _bfly_min_i32(e, es_v, lane_iota):
    for off in (1, 2, 4, 8):
        es_v[...] = e
        oe = plsc.load_gather(es_v, [lane_iota ^ off])
        e = jnp.minimum(e, oe)
    return e


def _row_argmax(buf, vs_v, es_v, lane_iota):
    neg_inf = jnp.full((LANES,), -jnp.inf, dtype=jnp.float32)
    zeros = jnp.zeros((LANES,), jnp.int32)

    def blk_step(b, carry):
        gmax, gblk = carry
        off = b * (BLK * LANES)
        vs = [buf[pl.ds(off + k * LANES, LANES)] for k in range(BLK)]
        while len(vs) > 1:
            vs = [jnp.maximum(vs[2 * i], vs[2 * i + 1])
                  for i in range(len(vs) // 2)]
        m = vs[0]
        pred = m > gmax
        gmax = jnp.where(pred, m, gmax)
        gblk = jnp.where(pred, jnp.full((LANES,), b, jnp.int32), gblk)
        return gmax, gblk

    gmax, gblk = lax.fori_loop(0, NBLK, blk_step, (neg_inf, zeros), unroll=4)

    mvec = _bfly_max(gmax, vs_v, lane_iota)
    cand = jnp.where(gmax == mvec, gblk, BIG)
    bsel = _bfly_min_i32(cand, es_v, lane_iota)
    bsel_s = bsel[0]

    off = bsel_s * (BLK * LANES)
    bp = jnp.full((LANES,), BIG, jnp.int32)
    for k in range(BLK):
        v = buf[pl.ds(off + k * LANES, LANES)]
        hit = v == mvec
        bp = jnp.minimum(bp, jnp.where(hit, jnp.int32(k), BIG))

    e = ((bsel * BLK + bp) * LANES) + lane_iota
    e = jnp.where(bp < BIG, e, INT_MAX)
    return _bfly_min_i32(e, es_v, lane_iota)


def _argmax_body(x_hbm, out_hbm, buf0, buf1, res_v, vs_v, es_v, sem0, sem1):
    c = lax.axis_index("c")
    s = lax.axis_index("s")
    wid = s * 2 + c
    base = wid * ROWS_PER_WORKER

    lane_iota = lax.iota(jnp.int32, LANES)

    def compute(buf, r_local):
        e = _row_argmax(buf, vs_v, es_v, lane_iota)
        goff = (r_local // LANES) * LANES
        cur = res_v[pl.ds(goff, LANES)]
        res_v[pl.ds(goff, LANES)] = jnp.where(
            lane_iota == (r_local - goff), e, cur)

    pltpu.make_async_copy(x_hbm.at[base], buf0, sem0).start()

    def pair(p, _):
        r0 = 2 * p
        pltpu.make_async_copy(x_hbm.at[base + r0 + 1], buf1, sem1).start()
        pltpu.make_async_copy(x_hbm.at[base + r0], buf0, sem0).wait()
        compute(buf0, r0)

        @pl.when(p < ROWS_PER_WORKER // 2 - 1)
        def _():
            pltpu.make_async_copy(x_hbm.at[base + r0 + 2], buf0, sem0).start()

        pltpu.make_async_copy(x_hbm.at[base + r0 + 1], buf1, sem1).wait()
        compute(buf1, r0 + 1)
        return 0

    lax.fori_loop(0, ROWS_PER_WORKER // 2, pair, 0)

    pltpu.sync_copy(res_v, out_hbm.at[pl.ds(base, ROWS_PER_WORKER)])


@jax.jit
def _argmax_rows(x):
    mesh = plsc.VectorSubcoreMesh(core_axis_name="c", subcore_axis_name="s")
    kern = functools.partial(
        pl.kernel,
        mesh=mesh,
        compiler_params=pltpu.CompilerParams(needs_layout_passes=False),
        out_type=jax.ShapeDtypeStruct((ROWS,), jnp.int32),
        scratch_types=[
            pltpu.VMEM((KV,), jnp.float32),
            pltpu.VMEM((KV,), jnp.float32),
            pltpu.VMEM((ROWS_PER_WORKER,), jnp.int32),
            pltpu.VMEM((LANES,), jnp.float32),
            pltpu.VMEM((LANES,), jnp.int32),
            pltpu.SemaphoreType.DMA,
            pltpu.SemaphoreType.DMA,
        ],
    )(_argmax_body)
    return kern(x)


def kernel(xyz):
    x = xyz.reshape(ROWS, KV)
    out = _argmax_rows(x)
    return out.reshape(B, Q)
